# baseline (speedup 1.0000x reference)
import functools

import jax
import jax.numpy as jnp
from jax import lax
from jax.experimental import pallas as pl
from jax.experimental.pallas import tpu as pltpu
from jax.experimental.pallas import tpu_sc as plsc

N = 10000
E = 160000
NPAD = 10240
EPAD = 163840
NG = 128
F32 = jnp.float32

def _mesh():
    return plsc.VectorSubcoreMesh(core_axis_name="c", subcore_axis_name="s",
                                  num_cores=2, num_subcores=16)

_EPT = EPAD // 32
_CD = 512
_CP = 320
_RPT = NPAD // 16


def _fill(buf, rows, cols, val):
    v = jnp.full((16,), val, F32)

    def body(i, carry):
        for j in range(cols // 16):
            buf[i, pl.ds(j * 16, 16)] = v
        return carry

    lax.fori_loop(0, rows, body, 0)


_G = 128
_GPT = _EPT // _G
_GS, _GF = 24, 56


def _deg_body(dst_hbm, out_hbm, buf, dstb, acc, sem):
    del sem
    c = lax.axis_index("c")
    s = lax.axis_index("s")
    base_row = (c * 16 + s) * _GPT
    pltpu.sync_copy(dst_hbm.at[pl.ds(base_row, _GPT)], dstb)
    _fill(buf, _G, 16, 0.0)
    for r in range(_RPT // _G):
        pltpu.sync_copy(buf, acc.at[pl.ds(s * _RPT + r * _G, _G)])
    _fill(buf, _G, 16, 1.0)
    plsc.subcore_barrier()

    def body(i, carry):
        pltpu.sync_copy(buf, acc.at[dstb.at[i]], add=True)
        return carry

    lax.fori_loop(0, _GPT, body, 0)
    plsc.subcore_barrier()
    off = c * NPAD + s * _RPT
    pltpu.sync_copy(acc.at[pl.ds(s * _RPT, _RPT)], out_hbm.at[pl.ds(off, _RPT)])


def _deg(dst2):
    out = pl.kernel(
        _deg_body,
        out_type=jax.ShapeDtypeStruct((2 * NPAD, 16), F32),
        mesh=_mesh(),
        scratch_types=[
            pltpu.VMEM((_G, 16), F32),
            pltpu.VMEM((_GPT, _G), jnp.int32),
            pltpu.VMEM_SHARED((NPAD, 16), F32),
            pltpu.SemaphoreType.DMA,
        ],
    )(dst2)
    return out.reshape(2, NPAD, 16)


def _prop_body(src_hbm, dst_hbm, h0, h1, h2, h3, out_hbm, srcb, dstb,
               rows_a, rows_b, acc, sem_a, sem_b):
    c = lax.axis_index("c")
    s = lax.axis_index("s")
    n_g = jnp.where(c == 0, _GS, _GF)
    base_row = jnp.where(c == 0, s * _GS, 16 * _GS + s * _GF)
    pltpu.sync_copy(src_hbm.at[pl.ds(base_row, _GF)], srcb)
    pltpu.sync_copy(dst_hbm.at[pl.ds(base_row, _GF)], dstb)
    for ck, hk in enumerate((h0, h1, h2, h3)):
        _fill(rows_a, _G, 128, 0.0)
        for r in range(_RPT // _G):
            pltpu.sync_copy(rows_a, acc.at[pl.ds(s * _RPT + r * _G, _G)])
        plsc.subcore_barrier()
        pltpu.async_copy(hk.at[srcb.at[0]], rows_a, sem_a)

        def body(i, carry):
            pltpu.async_copy(hk.at[srcb.at[2 * i + 1]], rows_b, sem_b)
            pltpu.make_async_copy(hk.at[srcb.at[0]], rows_a, sem_a).wait()
            pltpu.sync_copy(rows_a, acc.at[dstb.at[2 * i]], add=True)
            nxt = jnp.minimum(2 * i + 2, n_g - 1)
            pltpu.async_copy(hk.at[srcb.at[nxt]], rows_a, sem_a)
            pltpu.make_async_copy(hk.at[srcb.at[0]], rows_b, sem_b).wait()
            pltpu.sync_copy(rows_b, acc.at[dstb.at[2 * i + 1]], add=True)
            return carry

        lax.fori_loop(0, n_g // 2, body, 0)
        pltpu.make_async_copy(hk.at[srcb.at[0]], rows_a, sem_a).wait()
        plsc.subcore_barrier()
        off = ck * 2 * NPAD + c * NPAD + s * _RPT
        pltpu.sync_copy(acc.at[pl.ds(s * _RPT, _RPT)],
                        out_hbm.at[pl.ds(off, _RPT)])
        plsc.subcore_barrier()


def _prop(src2, dst2, H):
    out = pl.kernel(
        _prop_body,
        out_type=jax.ShapeDtypeStruct((8 * NPAD, 128), F32),
        mesh=_mesh(),
        scratch_types=[
            pltpu.VMEM((_GF, _G), jnp.int32),
            pltpu.VMEM((_GF, _G), jnp.int32),
            pltpu.VMEM((_G, 128), F32),
            pltpu.VMEM((_G, 128), F32),
            pltpu.VMEM_SHARED((NPAD, 128), F32),
            pltpu.SemaphoreType.DMA,
            pltpu.SemaphoreType.DMA,
        ],
    )(src2, dst2, H[0], H[1], H[2], H[3])
    return out.reshape(4, 2, NPAD, 128)


def _k2_body(degp_ref, x_ref, dis_ref, xs_ref):
    deg = jnp.sum(degp_ref[...], axis=(0, 2)) + 1.0
    dis = lax.rsqrt(deg)
    dis_ref[...] = dis[None, None, :]
    xs_ref[...] = dis[:, None] * x_ref[...]


def _k2(degp, xp):
    return pl.pallas_call(
        _k2_body,
        grid=(NPAD // 512,),
        in_specs=[
            pl.BlockSpec((2, 512, 16), lambda m: (0, m, 0)),
            pl.BlockSpec((512, 256), lambda m: (m, 0)),
        ],
        out_specs=[
            pl.BlockSpec((1, 1, 512), lambda m: (m, 0, 0)),
            pl.BlockSpec((512, 256), lambda m: (m, 0)),
        ],
        out_shape=[
            jax.ShapeDtypeStruct((NPAD // 512, 1, 512), F32),
            jax.ShapeDtypeStruct((NPAD, 256), F32),
        ],
    )(degp, xp)


def _mm1_body(a_ref, b_ref, o_ref):
    o_ref[0] = jnp.dot(a_ref[...], b_ref[...], preferred_element_type=F32)


def _mm1(xs, W1):
    return pl.pallas_call(
        _mm1_body,
        grid=(NPAD // 512, 4),
        in_specs=[
            pl.BlockSpec((512, 256), lambda m, nc: (m, 0)),
            pl.BlockSpec((256, 128), lambda m, nc: (0, nc)),
        ],
        out_specs=pl.BlockSpec((1, 512, 128), lambda m, nc: (nc, m, 0)),
        out_shape=jax.ShapeDtypeStruct((4, NPAD, 128), F32),
    )(xs, W1)


def _k5a_body(p_ref, h_ref, dis_ref, b_ref, zs_ref):
    t = p_ref[0, 0] + p_ref[0, 1] + h_ref[0]
    dis = dis_ref[0, 0]
    z = jnp.maximum(dis[:, None] * t + b_ref[0], 0.0)
    zs_ref[0] = dis[:, None] * z


def _k5a(P, H, dis2d, b4):
    return pl.pallas_call(
        _k5a_body,
        grid=(4, NPAD // 512),
        in_specs=[
            pl.BlockSpec((1, 2, 512, 128), lambda kc, m: (kc, 0, m, 0)),
            pl.BlockSpec((1, 512, 128), lambda kc, m: (kc, m, 0)),
            pl.BlockSpec((1, 1, 512), lambda kc, m: (m, 0, 0)),
            pl.BlockSpec((1, 1, 128), lambda kc, m: (kc, 0, 0)),
        ],
        out_specs=pl.BlockSpec((1, 512, 128), lambda kc, m: (kc, m, 0)),
        out_shape=jax.ShapeDtypeStruct((4, NPAD, 128), F32),
    )(P, H, dis2d, b4)


def _mm2_body(a_ref, b_ref, o_ref):
    @pl.when(pl.program_id(2) == 0)
    def _():
        o_ref[...] = jnp.zeros_like(o_ref)

    o_ref[0] += jnp.dot(a_ref[0], b_ref[...], preferred_element_type=F32)


def _mm2(zs, W2):
    return pl.pallas_call(
        _mm2_body,
        grid=(NPAD // 512, 4, 4),
        in_specs=[
            pl.BlockSpec((1, 512, 128), lambda m, nc, kc: (kc, m, 0)),
            pl.BlockSpec((128, 128), lambda m, nc, kc: (kc, nc)),
        ],
        out_specs=pl.BlockSpec((1, 512, 128), lambda m, nc, kc: (nc, m, 0)),
        out_shape=jax.ShapeDtypeStruct((4, NPAD, 128), F32),
    )(zs, W2)


def _k7a_body(p_ref, h_ref, dis_ref, b_ref, bat_ref, s_ref, cnt_ref):
    kc = pl.program_id(0)
    m = pl.program_id(1)
    t = p_ref[0, 0] + p_ref[0, 1] + h_ref[0]
    dis = dis_ref[0, 0]
    z = jnp.maximum(dis[:, None] * t + b_ref[0], 0.0)
    oh = (lax.broadcasted_iota(jnp.int32, (128, 512), 0)
          == bat_ref[0, 0][None, :]).astype(F32)

    @pl.when(m == 0)
    def _():
        s_ref[...] = jnp.zeros_like(s_ref)

    s_ref[...] += jnp.dot(oh, z, preferred_element_type=F32)

    @pl.when((m == 0) & (kc == 0))
    def _():
        cnt_ref[...] = jnp.zeros_like(cnt_ref)

    @pl.when(kc == 0)
    def _():
        cnt_ref[...] += jnp.sum(oh, axis=1)[:, None]


def _k7a(P, H, dis2d, b4, bat2d):
    return pl.pallas_call(
        _k7a_body,
        grid=(4, NPAD // 512),
        in_specs=[
            pl.BlockSpec((1, 2, 512, 128), lambda kc, m: (kc, 0, m, 0)),
            pl.BlockSpec((1, 512, 128), lambda kc, m: (kc, m, 0)),
            pl.BlockSpec((1, 1, 512), lambda kc, m: (m, 0, 0)),
            pl.BlockSpec((1, 1, 128), lambda kc, m: (kc, 0, 0)),
            pl.BlockSpec((1, 1, 512), lambda kc, m: (m, 0, 0)),
        ],
        out_specs=[
            pl.BlockSpec((128, 128), lambda kc, m: (0, kc)),
            pl.BlockSpec((128, 128), lambda kc, m: (0, 0)),
        ],
        out_shape=[
            jax.ShapeDtypeStruct((128, 512), F32),
            jax.ShapeDtypeStruct((128, 128), F32),
        ],
    )(P, H, dis2d, b4, bat2d)


def _k7b_body(s_ref, cnt_ref, w_ref, b_ref, o_ref):
    pooled = s_ref[...] / jnp.maximum(cnt_ref[:, 0:1], 1.0)
    o_ref[...] = jnp.dot(pooled, w_ref[...], preferred_element_type=F32) \
        + b_ref[...]


def _k7b(S, cnt, W3, b3r):
    return pl.pallas_call(
        _k7b_body,
        out_shape=jax.ShapeDtypeStruct((128, 256), F32),
    )(S, cnt, W3, b3r)


def kernel(x, edge_index, batch, W1, b1, W2, b2, W3, b3):
    src = jnp.concatenate(
        [edge_index[0], jnp.zeros((EPAD - E,), jnp.int32)]).reshape(-1, 128)
    pad_dst = N + jnp.arange(EPAD - E, dtype=jnp.int32) % (NPAD - N)
    dst = jnp.concatenate([edge_index[1], pad_dst]).reshape(-1, 128)
    xp = jnp.pad(x, ((0, NPAD - N), (0, 0)))
    bat2d = jnp.pad(batch, (0, NPAD - N),
                    constant_values=NG).reshape(NPAD // 512, 1, 512)

    degp = _deg(dst)
    dis2d, xs = _k2(degp, xp)
    H0 = _mm1(xs, W1)
    P1 = _prop(src, dst, H0)
    zs1 = _k5a(P1, H0, dis2d, b1.reshape(4, 1, 128))
    H1 = _mm2(zs1, W2)
    P2 = _prop(src, dst, H1)
    S, cnt = _k7a(P2, H1, dis2d, b2.reshape(4, 1, 128), bat2d)
    return _k7b(S, cnt, W3, b3.reshape(1, 256))


# speedup vs baseline: 1.1974x; 1.1974x over previous
import functools

import jax
import jax.numpy as jnp
from jax import lax
from jax.experimental import pallas as pl
from jax.experimental.pallas import tpu as pltpu
from jax.experimental.pallas import tpu_sc as plsc

N = 10000
E = 160000
NPAD = 10240
EPAD = 163840
NG = 128
F32 = jnp.float32

def _mesh():
    return plsc.VectorSubcoreMesh(core_axis_name="c", subcore_axis_name="s",
                                  num_cores=2, num_subcores=16)

_EPT = EPAD // 32
_CD = 512
_CP = 320
_RPT = NPAD // 16


def _fill(buf, rows, cols, val):
    v = jnp.full((16,), val, F32)

    def body(i, carry):
        for j in range(cols // 16):
            buf[i, pl.ds(j * 16, 16)] = v
        return carry

    lax.fori_loop(0, rows, body, 0)


_G = 128
_GPT = _EPT // _G


def _deg_body(dst_hbm, out_hbm, buf, dstb, acc, sem):
    del sem
    c = lax.axis_index("c")
    s = lax.axis_index("s")
    base_row = (c * 16 + s) * _GPT
    pltpu.sync_copy(dst_hbm.at[pl.ds(base_row, _GPT)], dstb)
    _fill(buf, _G, 16, 0.0)
    for r in range(_RPT // _G):
        pltpu.sync_copy(buf, acc.at[pl.ds(s * _RPT + r * _G, _G)])
    _fill(buf, _G, 16, 1.0)
    plsc.subcore_barrier()

    def body(i, carry):
        pltpu.sync_copy(buf, acc.at[dstb.at[i]], add=True)
        return carry

    lax.fori_loop(0, _GPT, body, 0)
    plsc.subcore_barrier()
    off = c * NPAD + s * _RPT
    pltpu.sync_copy(acc.at[pl.ds(s * _RPT, _RPT)], out_hbm.at[pl.ds(off, _RPT)])


def _deg(dst2):
    out = pl.kernel(
        _deg_body,
        out_type=jax.ShapeDtypeStruct((2 * NPAD, 16), F32),
        mesh=_mesh(),
        scratch_types=[
            pltpu.VMEM((_G, 16), F32),
            pltpu.VMEM((_GPT, _G), jnp.int32),
            pltpu.VMEM_SHARED((NPAD, 16), F32),
            pltpu.SemaphoreType.DMA,
        ],
    )(dst2)
    return out.reshape(2, NPAD, 16)


def _prop_body(src_hbm, dst_hbm, h0, h1, h2, h3, out_hbm, srcb, dstb,
               rows_a, rows_b, acc, sem_a, sem_b):
    c = lax.axis_index("c")
    s = lax.axis_index("s")
    base_row = (c * 16 + s) * _GPT
    pltpu.sync_copy(src_hbm.at[pl.ds(base_row, _GPT)], srcb)
    pltpu.sync_copy(dst_hbm.at[pl.ds(base_row, _GPT)], dstb)
    for ck, hk in enumerate((h0, h1, h2, h3)):
        _fill(rows_a, _G, 128, 0.0)
        for r in range(_RPT // _G):
            pltpu.sync_copy(rows_a, acc.at[pl.ds(s * _RPT + r * _G, _G)])
        plsc.subcore_barrier()
        pltpu.async_copy(hk.at[srcb.at[0]], rows_a, sem_a)

        def body(i, carry):
            pltpu.async_copy(hk.at[srcb.at[2 * i + 1]], rows_b, sem_b)
            pltpu.make_async_copy(hk.at[srcb.at[0]], rows_a, sem_a).wait()
            pltpu.sync_copy(rows_a, acc.at[dstb.at[2 * i]], add=True)
            nxt = jnp.minimum(2 * i + 2, _GPT - 1)
            pltpu.async_copy(hk.at[srcb.at[nxt]], rows_a, sem_a)
            pltpu.make_async_copy(hk.at[srcb.at[0]], rows_b, sem_b).wait()
            pltpu.sync_copy(rows_b, acc.at[dstb.at[2 * i + 1]], add=True)
            return carry

        lax.fori_loop(0, _GPT // 2, body, 0)
        pltpu.make_async_copy(hk.at[srcb.at[0]], rows_a, sem_a).wait()
        plsc.subcore_barrier()
        off = ck * 2 * NPAD + c * NPAD + s * _RPT
        pltpu.sync_copy(acc.at[pl.ds(s * _RPT, _RPT)],
                        out_hbm.at[pl.ds(off, _RPT)])
        plsc.subcore_barrier()


def _prop(src2, dst2, H):
    out = pl.kernel(
        _prop_body,
        out_type=jax.ShapeDtypeStruct((8 * NPAD, 128), F32),
        mesh=_mesh(),
        scratch_types=[
            pltpu.VMEM((_GPT, _G), jnp.int32),
            pltpu.VMEM((_GPT, _G), jnp.int32),
            pltpu.VMEM((_G, 128), F32),
            pltpu.VMEM((_G, 128), F32),
            pltpu.VMEM_SHARED((NPAD, 128), F32),
            pltpu.SemaphoreType.DMA,
            pltpu.SemaphoreType.DMA,
        ],
    )(src2, dst2, H[0], H[1], H[2], H[3])
    return out.reshape(4, 2, NPAD, 128)


def _mm1f_body(degp_ref, x_ref, w_ref, dis_ref, o_ref):
    deg = jnp.sum(degp_ref[...], axis=(0, 2)) + 1.0
    dis = lax.rsqrt(deg)
    dis_ref[...] = dis[None, None, :]
    xs = dis[:, None] * x_ref[...]
    r = jnp.dot(xs, w_ref[...], preferred_element_type=F32)
    for nc in range(4):
        o_ref[nc] = r[:, nc * 128:(nc + 1) * 128]


def _mm1f(degp, xp, W1):
    return pl.pallas_call(
        _mm1f_body,
        grid=(NPAD // 512,),
        in_specs=[
            pl.BlockSpec((2, 512, 16), lambda m: (0, m, 0)),
            pl.BlockSpec((512, 256), lambda m: (m, 0)),
            pl.BlockSpec((256, 512), lambda m: (0, 0)),
        ],
        out_specs=[
            pl.BlockSpec((1, 1, 512), lambda m: (m, 0, 0)),
            pl.BlockSpec((4, 512, 128), lambda m: (0, m, 0)),
        ],
        out_shape=[
            jax.ShapeDtypeStruct((NPAD // 512, 1, 512), F32),
            jax.ShapeDtypeStruct((4, NPAD, 128), F32),
        ],
    )(degp, xp, W1)


def _mm2f_body(p_ref, h_ref, dis_ref, b_ref, w_ref, o_ref):
    kc = pl.program_id(1)
    t = p_ref[0, 0] + p_ref[0, 1] + h_ref[0]
    dis = dis_ref[0, 0]
    zs = dis[:, None] * jnp.maximum(dis[:, None] * t + b_ref[0, 0], 0.0)
    r = jnp.dot(zs, w_ref[0], preferred_element_type=F32)

    @pl.when(kc == 0)
    def _():
        o_ref[...] = jnp.zeros_like(o_ref)

    for nc in range(4):
        o_ref[nc] += r[:, nc * 128:(nc + 1) * 128]


def _mm2f(P, H, dis2d, b4, W2r):
    return pl.pallas_call(
        _mm2f_body,
        grid=(NPAD // 512, 4),
        in_specs=[
            pl.BlockSpec((1, 2, 512, 128), lambda m, kc: (kc, 0, m, 0)),
            pl.BlockSpec((1, 512, 128), lambda m, kc: (kc, m, 0)),
            pl.BlockSpec((1, 1, 512), lambda m, kc: (m, 0, 0)),
            pl.BlockSpec((1, 1, 128), lambda m, kc: (kc, 0, 0)),
            pl.BlockSpec((1, 128, 512), lambda m, kc: (kc, 0, 0)),
        ],
        out_specs=pl.BlockSpec((4, 512, 128), lambda m, kc: (0, m, 0)),
        out_shape=jax.ShapeDtypeStruct((4, NPAD, 128), F32),
    )(P, H, dis2d, b4, W2r)


def _k7a_body(p_ref, h_ref, dis_ref, b_ref, bat_ref, s_ref, cnt_ref):
    kc = pl.program_id(0)
    m = pl.program_id(1)
    t = p_ref[0, 0] + p_ref[0, 1] + h_ref[0]
    dis = dis_ref[0, 0]
    z = jnp.maximum(dis[:, None] * t + b_ref[0], 0.0)
    oh = (lax.broadcasted_iota(jnp.int32, (128, 512), 0)
          == bat_ref[0, 0][None, :]).astype(F32)

    @pl.when(m == 0)
    def _():
        s_ref[...] = jnp.zeros_like(s_ref)

    s_ref[...] += jnp.dot(oh, z, preferred_element_type=F32)

    @pl.when((m == 0) & (kc == 0))
    def _():
        cnt_ref[...] = jnp.zeros_like(cnt_ref)

    @pl.when(kc == 0)
    def _():
        cnt_ref[...] += jnp.sum(oh, axis=1)[:, None]


def _k7a(P, H, dis2d, b4, bat2d):
    return pl.pallas_call(
        _k7a_body,
        grid=(4, NPAD // 512),
        in_specs=[
            pl.BlockSpec((1, 2, 512, 128), lambda kc, m: (kc, 0, m, 0)),
            pl.BlockSpec((1, 512, 128), lambda kc, m: (kc, m, 0)),
            pl.BlockSpec((1, 1, 512), lambda kc, m: (m, 0, 0)),
            pl.BlockSpec((1, 1, 128), lambda kc, m: (kc, 0, 0)),
            pl.BlockSpec((1, 1, 512), lambda kc, m: (m, 0, 0)),
        ],
        out_specs=[
            pl.BlockSpec((128, 128), lambda kc, m: (0, kc)),
            pl.BlockSpec((128, 128), lambda kc, m: (0, 0)),
        ],
        out_shape=[
            jax.ShapeDtypeStruct((128, 512), F32),
            jax.ShapeDtypeStruct((128, 128), F32),
        ],
    )(P, H, dis2d, b4, bat2d)


def _k7b_body(s_ref, cnt_ref, w_ref, b_ref, o_ref):
    pooled = s_ref[...] / jnp.maximum(cnt_ref[:, 0:1], 1.0)
    o_ref[...] = jnp.dot(pooled, w_ref[...], preferred_element_type=F32) \
        + b_ref[...]


def _k7b(S, cnt, W3, b3r):
    return pl.pallas_call(
        _k7b_body,
        out_shape=jax.ShapeDtypeStruct((128, 256), F32),
    )(S, cnt, W3, b3r)


def kernel(x, edge_index, batch, W1, b1, W2, b2, W3, b3):
    src = jnp.concatenate(
        [edge_index[0], jnp.zeros((EPAD - E,), jnp.int32)]).reshape(-1, 128)
    pad_dst = N + jnp.arange(EPAD - E, dtype=jnp.int32) % (NPAD - N)
    dst = jnp.concatenate([edge_index[1], pad_dst]).reshape(-1, 128)
    xp = jnp.pad(x, ((0, NPAD - N), (0, 0)))
    bat2d = jnp.pad(batch, (0, NPAD - N),
                    constant_values=NG).reshape(NPAD // 512, 1, 512)

    degp = _deg(dst)
    dis2d, H0 = _mm1f(degp, xp, W1)
    P1 = _prop(src, dst, H0)
    H1 = _mm2f(P1, H0, dis2d, b1.reshape(4, 1, 128), W2.reshape(4, 128, 512))
    P2 = _prop(src, dst, H1)
    S, cnt = _k7a(P2, H1, dis2d, b2.reshape(4, 1, 128), bat2d)
    return _k7b(S, cnt, W3, b3.reshape(1, 256))
